# baseline (speedup 1.0000x reference)
import jax
import jax.numpy as jnp
from jax import lax
from jax.experimental import pallas as pl
from jax.experimental.pallas import tpu as pltpu

E = 8
K = 2
BN = 4096


def _fused_body(x_ref, w_ref, b_ref, g_ref, i_ref):
    lg = lax.dot_general(
        x_ref[:, :], w_ref[:, :],
        dimension_numbers=(((1,), (1,)), ((), ())),
        preferred_element_type=jnp.float32,
    ) + b_ref[:, :]
    it = lax.broadcasted_iota(jnp.int32, (BN, E), 1)
    m1 = jnp.max(lg, axis=1, keepdims=True)
    a1 = jnp.min(jnp.where(lg == m1, it, E), axis=1, keepdims=True)
    cand = jnp.where(it == a1, -jnp.inf, lg)
    m2 = jnp.max(cand, axis=1, keepdims=True)
    a2 = jnp.min(jnp.where(cand == m2, it, E), axis=1, keepdims=True)
    s = jnp.exp(m2 - m1)
    inv = 1.0 / (1.0 + s)
    g_ref[:, 0:1] = inv
    g_ref[:, 1:2] = s * inv
    i_ref[:, 0:1] = a1
    i_ref[:, 1:2] = a2


def kernel(x, W, b):
    n, d = x.shape
    grid = n // BN
    return pl.pallas_call(
        _fused_body,
        grid=(grid,),
        in_specs=[
            pl.BlockSpec((BN, d), lambda i: (i, 0)),
            pl.BlockSpec((E, d), lambda i: (0, 0)),
            pl.BlockSpec((1, E), lambda i: (0, 0)),
        ],
        out_specs=[
            pl.BlockSpec((BN, K), lambda i: (i, 0)),
            pl.BlockSpec((BN, K), lambda i: (i, 0)),
        ],
        out_shape=[
            jax.ShapeDtypeStruct((n, K), jnp.float32),
            jax.ShapeDtypeStruct((n, K), jnp.int32),
        ],
        compiler_params=pltpu.CompilerParams(
            dimension_semantics=("parallel",),
        ),
    )(x, W, b.reshape(1, E))


# speedup vs baseline: 2.0303x; 2.0303x over previous
import functools

import jax
import jax.numpy as jnp
from jax import lax
from jax.experimental import pallas as pl
from jax.experimental.pallas import tpu as pltpu
from jax.experimental.pallas import tpu_sc as plsc

E = 8
K = 2
BN = 4096


def _logits_body(x_ref, w_ref, b_ref, out_ref):
    out_ref[:, :] = lax.dot_general(
        w_ref[:, :], x_ref[:, :],
        dimension_numbers=(((1,), (1,)), ((), ())),
        preferred_element_type=jnp.float32,
    ) + b_ref[:, :]


def _logits_t_chunk(x, W, b2, block_base, nc):
    d = x.shape[1]
    grid = nc // BN
    return pl.pallas_call(
        _logits_body,
        grid=(grid,),
        in_specs=[
            pl.BlockSpec((BN, d), lambda i: (block_base + i, 0)),
            pl.BlockSpec((E, d), lambda i: (0, 0)),
            pl.BlockSpec((E, 1), lambda i: (0, 0)),
        ],
        out_specs=pl.BlockSpec((E, BN), lambda i: (0, i)),
        out_shape=jax.ShapeDtypeStruct((E, nc), jnp.float32),
        compiler_params=pltpu.CompilerParams(
            dimension_semantics=("parallel",),
        ),
    )(x, W, b2)


def _fused_body(x_ref, w_ref, b_ref, g_ref, i_ref):
    lg = lax.dot_general(
        w_ref[:, :], x_ref[:, :],
        dimension_numbers=(((1,), (1,)), ((), ())),
        preferred_element_type=jnp.float32,
    ) + jnp.transpose(b_ref[:, :])
    it = lax.broadcasted_iota(jnp.int32, (E, BN), 0)
    m1 = jnp.max(lg, axis=0, keepdims=True)
    a1 = jnp.min(jnp.where(lg == m1, it, E), axis=0, keepdims=True)
    cand = jnp.where(it == a1, -jnp.inf, lg)
    m2 = jnp.max(cand, axis=0, keepdims=True)
    a2 = jnp.min(jnp.where(cand == m2, it, E), axis=0, keepdims=True)
    s = jnp.exp(m2 - m1)
    inv = 1.0 / (1.0 + s)
    g_ref[0:1, :] = inv
    g_ref[1:2, :] = s * inv
    i_ref[0:1, :] = a1
    i_ref[1:2, :] = a2


def _routed_chunk_tc(x, W, b2, block_base, nc):
    d = x.shape[1]
    grid = nc // BN
    return pl.pallas_call(
        _fused_body,
        grid=(grid,),
        in_specs=[
            pl.BlockSpec((BN, d), lambda i: (block_base + i, 0)),
            pl.BlockSpec((E, d), lambda i: (0, 0)),
            pl.BlockSpec((1, E), lambda i: (0, 0)),
        ],
        out_specs=[
            pl.BlockSpec((K, BN), lambda i: (0, i)),
            pl.BlockSpec((K, BN), lambda i: (0, i)),
        ],
        out_shape=[
            jax.ShapeDtypeStruct((K, nc), jnp.float32),
            jax.ShapeDtypeStruct((K, nc), jnp.int32),
        ],
        compiler_params=pltpu.CompilerParams(
            dimension_semantics=("parallel",),
        ),
    )(x, W, b2)


def _make_router(nc):
    nw = 32
    tpw = nc // nw

    @functools.partial(
        pl.kernel,
        out_type=[
            jax.ShapeDtypeStruct((K, nc), jnp.float32),
            jax.ShapeDtypeStruct((K, nc), jnp.int32),
        ],
        mesh=plsc.VectorSubcoreMesh(core_axis_name="c", subcore_axis_name="s"),
        scratch_types=[
            pltpu.VMEM((E, tpw), jnp.float32),
            pltpu.VMEM((K, tpw), jnp.float32),
            pltpu.VMEM((K, tpw), jnp.int32),
        ],
    )
    def router(logits_hbm, gates_hbm, idx_hbm, lv, gv, iv):
        wid = lax.axis_index("s") * 2 + lax.axis_index("c")
        base = wid * tpw
        pltpu.sync_copy(logits_hbm.at[:, pl.ds(base, tpw)], lv)

        neg = jnp.full((16,), -1e30, jnp.float32)

        def body(g, carry):
            t = g * 16
            v = [lv[e, pl.ds(t, 16)] for e in range(E)]
            m1 = v[0]
            a1 = jnp.zeros((16,), jnp.int32)
            for e in range(1, E):
                gt = v[e] > m1
                m1 = jnp.where(gt, v[e], m1)
                a1 = jnp.where(gt, jnp.full((16,), e, jnp.int32), a1)
            m2 = neg
            a2 = jnp.zeros((16,), jnp.int32)
            for e in range(E):
                ev = jnp.full((16,), e, jnp.int32)
                cand = jnp.where(a1 == ev, neg, v[e])
                gt = cand > m2
                m2 = jnp.where(gt, cand, m2)
                a2 = jnp.where(gt, ev, a2)
            s = jnp.exp(m2 - m1)
            inv = 1.0 / (1.0 + s)
            gv[0, pl.ds(t, 16)] = inv
            gv[1, pl.ds(t, 16)] = s * inv
            iv[0, pl.ds(t, 16)] = a1
            iv[1, pl.ds(t, 16)] = a2
            return carry

        lax.fori_loop(0, tpw // 16, body, 0)
        pltpu.sync_copy(gv, gates_hbm.at[:, pl.ds(base, tpw)])
        pltpu.sync_copy(iv, idx_hbm.at[:, pl.ds(base, tpw)])

    return router


def kernel(x, W, b):
    n = x.shape[0]
    b2 = b.reshape(1, E)
    g1, i1 = _routed_chunk_tc(x, W, b2, 0, n)
    return g1.T, i1.T
